# baseline (speedup 1.0000x reference)
import numpy as np
import jax
import jax.numpy as jnp
from jax.experimental import pallas as pl
from jax.experimental.pallas import tpu as pltpu

D_MODEL = 512
N_HEADS = 8
HEAD_DIM = D_MODEL // N_HEADS
N_ROPE = 32
N_RFF = 32
POS_TAIL = 16
N_ROT = 16
POS_SCALE = 0.1
RMS_EPS = 1e-6
NF = 128

_j = np.arange(D_MODEL) % HEAD_DIM
_i = np.minimum(_j // 2, N_ROT - 1)
_p = np.where(_j >= HEAD_DIM - POS_TAIL, _j - (HEAD_DIM - POS_TAIL), 0)
_even = (_j < 2 * N_ROT) & (_j % 2 == 0)
_odd = (_j < 2 * N_ROT) & (_j % 2 == 1)
_mid = (_j >= 2 * N_ROT) & (_j < HEAD_DIM - POS_TAIL)
_tail = _j >= HEAD_DIM - POS_TAIL


def _static_g():
    G = np.zeros((NF, 4 * D_MODEL), np.float32)
    c = np.arange(D_MODEL)
    G[_i[_even], c[_even]] = 1.0
    G[16 + _i[_odd], c[_odd]] = 1.0
    G[80, c[_mid]] = 1.0
    G[32 + _i[_even], D_MODEL + c[_even]] = 1.0
    G[48 + _i[_odd], 2 * D_MODEL + c[_odd]] = 1.0
    G[64 + _p[_tail], 3 * D_MODEL + c[_tail]] = 1.0
    return G


_G_STATIC = _static_g()


def _fused_kernel(x_ref, spk_ref, valid_ref, f_ref, g_ref, gv_ref, ones_ref,
                  w_ref, wq_ref, wk_ref, wv_ref, wo_ref, o_ref):
    D = D_MODEL
    x = x_ref[0, 0]
    w = w_ref[0]
    cdims = (((1,), (1,)), ((), ()))
    kdims = (((1,), (0,)), ((), ()))

    x2 = (x * x).astype(jnp.bfloat16)
    ssum = jax.lax.dot_general(x2, ones_ref[...], kdims,
                               preferred_element_type=jnp.float32)[:, 0:1]
    xn = x * jax.lax.rsqrt(ssum * (1.0 / D) + RMS_EPS)
    xn = xn * w[None, :]

    xnb = xn.astype(jnp.bfloat16)
    q = jax.lax.dot_general(xnb, wq_ref[...], cdims, preferred_element_type=jnp.float32)
    k = jax.lax.dot_general(xnb, wk_ref[...], cdims, preferred_element_type=jnp.float32)
    v = jax.lax.dot_general(xnb, wv_ref[...], cdims, preferred_element_type=jnp.float32)

    maps = jax.lax.dot_general(f_ref[0], g_ref[...], kdims,
                               preferred_element_type=jnp.float32)
    A = maps[:, 0:D]
    Bm = maps[:, D:2 * D]
    Cm = maps[:, 2 * D:3 * D]
    kt = maps[:, 3 * D:4 * D]

    def rot(t, tailscale):
        tl = jnp.concatenate([t[:, 1:], t[:, :1]], axis=1)
        tr = jnp.concatenate([t[:, -1:], t[:, :-1]], axis=1)
        return A * t + Bm * tl + Cm * tr + tailscale * kt

    q = rot(q, gv_ref[0])
    k = rot(k, 1.0)

    spk = spk_ref[0, 0]
    madd = (spk - 1.0) * 1e30
    haskey = jnp.max(spk)

    qb = q.astype(jnp.bfloat16)
    kb = k.astype(jnp.bfloat16)
    vb = v.astype(jnp.bfloat16)
    outs = []
    for h in range(N_HEADS):
        sl = slice(h * HEAD_DIM, (h + 1) * HEAD_DIM)
        s = jax.lax.dot_general(qb[:, sl], kb[:, sl], cdims,
                                preferred_element_type=jnp.float32)
        e = jnp.exp(s + madd)
        r = haskey / (jnp.sum(e, axis=-1, keepdims=True) + 1e-37)
        oh = jnp.dot(e.astype(jnp.bfloat16), vb[:, sl],
                     preferred_element_type=jnp.float32)
        outs.append(oh * r)
    oc = jnp.concatenate(outs, axis=1)

    y = jax.lax.dot_general(oc.astype(jnp.bfloat16), wo_ref[...], cdims,
                            preferred_element_type=jnp.float32)
    y = y * valid_ref[0]
    o_ref[0, 0] = y


def kernel(x, point_positions, neuron_pad_mask, spike_mask, Wq, Wk, Wv, Wo, rms_w,
           rope_dirs, rope_freqs, rff_Omega, posC_W, pos_head_gain):
    B, T, N, D = x.shape
    scale = 1.0 / np.sqrt(HEAD_DIM)

    theta = jnp.einsum('bnd,fd->bnf', point_positions, rope_dirs) * rope_freqs
    theta = theta[..., :N_ROT]
    ct = jnp.cos(theta)
    st = jnp.sin(theta)
    proj = jnp.einsum('bnd,md->bnm', point_positions, rff_Omega)
    phi = jnp.concatenate([jnp.cos(proj), jnp.sin(proj)], axis=-1)
    pos_feat = jnp.einsum('bnm,pm->bnp', phi, posC_W)
    ones_bn = jnp.ones((B, N, 1), jnp.float32)
    F = jnp.concatenate(
        [ct, ct - st * st, -st, ct * st, pos_feat, ones_bn,
         jnp.zeros((B, N, NF - 81), jnp.float32)], axis=-1).astype(jnp.bfloat16)

    G = jnp.asarray(_G_STATIC).astype(jnp.bfloat16)
    gvec = jnp.concatenate(
        [jnp.zeros((N_HEADS, HEAD_DIM - POS_TAIL), jnp.float32),
         POS_SCALE * scale * pos_head_gain], axis=1).reshape(1, D)

    valid = neuron_pad_mask != 0
    spk = ((spike_mask != 0) & valid[:, None, :]).astype(jnp.float32)
    spk4 = spk.reshape(B, T, 1, N)
    validf = valid.astype(jnp.float32).reshape(B, N, 1)
    rw = rms_w.reshape(1, D)
    ones_mx = jnp.ones((N, 128), jnp.bfloat16)

    wqb = (Wq * scale).astype(jnp.bfloat16)
    wkb = Wk.astype(jnp.bfloat16)
    wvb = Wv.astype(jnp.bfloat16)
    wob = Wo.astype(jnp.bfloat16)

    out = pl.pallas_call(
        _fused_kernel,
        grid=(B, T),
        in_specs=[
            pl.BlockSpec((1, 1, N, D), lambda b, t: (b, t, 0, 0)),
            pl.BlockSpec((1, 1, 1, N), lambda b, t: (b, t, 0, 0)),
            pl.BlockSpec((1, N, 1), lambda b, t: (b, 0, 0)),
            pl.BlockSpec((1, N, NF), lambda b, t: (b, 0, 0)),
            pl.BlockSpec((NF, 4 * D), lambda b, t: (0, 0)),
            pl.BlockSpec((1, D), lambda b, t: (0, 0)),
            pl.BlockSpec((N, 128), lambda b, t: (0, 0)),
            pl.BlockSpec((1, D), lambda b, t: (0, 0)),
            pl.BlockSpec((D, D), lambda b, t: (0, 0)),
            pl.BlockSpec((D, D), lambda b, t: (0, 0)),
            pl.BlockSpec((D, D), lambda b, t: (0, 0)),
            pl.BlockSpec((D, D), lambda b, t: (0, 0)),
        ],
        out_specs=pl.BlockSpec((1, 1, N, D), lambda b, t: (b, t, 0, 0)),
        out_shape=jax.ShapeDtypeStruct((B, T, N, D), jnp.float32),
        compiler_params=pltpu.CompilerParams(
            dimension_semantics=("parallel", "parallel")),
    )(x, spk4, validf, F, G, gvec, ones_mx, rw, wqb, wkb, wvb, wob)
    return out


# speedup vs baseline: 1.0685x; 1.0685x over previous
import numpy as np
import jax
import jax.numpy as jnp
from jax.experimental import pallas as pl
from jax.experimental.pallas import tpu as pltpu

D_MODEL = 512
N_HEADS = 8
HEAD_DIM = D_MODEL // N_HEADS
N_ROPE = 32
N_RFF = 32
POS_TAIL = 16
N_ROT = 16
POS_SCALE = 0.1
RMS_EPS = 1e-6
NF = 128

_j = np.arange(D_MODEL) % HEAD_DIM
_i = np.minimum(_j // 2, N_ROT - 1)
_p = np.where(_j >= HEAD_DIM - POS_TAIL, _j - (HEAD_DIM - POS_TAIL), 0)
_even = (_j < 2 * N_ROT) & (_j % 2 == 0)
_odd = (_j < 2 * N_ROT) & (_j % 2 == 1)
_mid = (_j >= 2 * N_ROT) & (_j < HEAD_DIM - POS_TAIL)
_tail = _j >= HEAD_DIM - POS_TAIL


def _static_g():
    G = np.zeros((NF, 4 * D_MODEL), np.float32)
    c = np.arange(D_MODEL)
    G[_i[_even], c[_even]] = 1.0
    G[16 + _i[_odd], c[_odd]] = 1.0
    G[80, c[_mid]] = 1.0
    G[32 + _i[_even], D_MODEL + c[_even]] = 1.0
    G[48 + _i[_odd], 2 * D_MODEL + c[_odd]] = 1.0
    G[64 + _p[_tail], 3 * D_MODEL + c[_tail]] = 1.0
    return G


_G_STATIC = _static_g()


def _fused_kernel(x_ref, spk_ref, valid_ref, f_ref, g_ref, gv_ref, ones_ref,
                  w_ref, wqkv_ref, wo_ref, o_ref):
    D = D_MODEL
    x = x_ref[0, 0]
    w = w_ref[0]
    cdims = (((1,), (1,)), ((), ()))
    kdims = (((1,), (0,)), ((), ()))

    x2 = (x * x).astype(jnp.bfloat16)
    ssum = jax.lax.dot_general(x2, ones_ref[...], kdims,
                               preferred_element_type=jnp.float32)[:, 0:1]
    xn = x * jax.lax.rsqrt(ssum * (1.0 / D) + RMS_EPS)
    xn = xn * w[None, :]

    xnb = xn.astype(jnp.bfloat16)
    qkv = jax.lax.dot_general(xnb, wqkv_ref[...], cdims,
                              preferred_element_type=jnp.float32)
    q = qkv[:, 0:D]
    k = qkv[:, D:2 * D]
    v = qkv[:, 2 * D:3 * D]

    maps = jax.lax.dot_general(f_ref[0], g_ref[...], kdims,
                               preferred_element_type=jnp.float32)
    A = maps[:, 0:D]
    Bm = maps[:, D:2 * D]
    Cm = maps[:, 2 * D:3 * D]
    kt = maps[:, 3 * D:4 * D]

    def rot(t, tailscale):
        tl = jnp.concatenate([t[:, 1:], t[:, :1]], axis=1)
        tr = jnp.concatenate([t[:, -1:], t[:, :-1]], axis=1)
        return A * t + Bm * tl + Cm * tr + tailscale * kt

    q = rot(q, gv_ref[0])
    k = rot(k, 1.0)

    spk = spk_ref[0, 0]
    madd = (spk - 1.0) * 1e30
    haskey = jnp.max(spk)

    qb = q.astype(jnp.bfloat16)
    kb = k.astype(jnp.bfloat16)
    vb = v.astype(jnp.bfloat16)
    outs = []
    for h in range(N_HEADS):
        sl = slice(h * HEAD_DIM, (h + 1) * HEAD_DIM)
        s = jax.lax.dot_general(qb[:, sl], kb[:, sl], cdims,
                                preferred_element_type=jnp.float32)
        e = jnp.exp(s + madd)
        r = haskey / (jnp.sum(e, axis=-1, keepdims=True) + 1e-37)
        oh = jnp.dot(e.astype(jnp.bfloat16), vb[:, sl],
                     preferred_element_type=jnp.float32)
        outs.append(oh * r)
    oc = jnp.concatenate(outs, axis=1)

    y = jax.lax.dot_general(oc.astype(jnp.bfloat16), wo_ref[...], cdims,
                            preferred_element_type=jnp.float32)
    y = y * valid_ref[0]
    o_ref[0, 0] = y


def kernel(x, point_positions, neuron_pad_mask, spike_mask, Wq, Wk, Wv, Wo, rms_w,
           rope_dirs, rope_freqs, rff_Omega, posC_W, pos_head_gain):
    B, T, N, D = x.shape
    scale = 1.0 / np.sqrt(HEAD_DIM)

    theta = jnp.einsum('bnd,fd->bnf', point_positions, rope_dirs) * rope_freqs
    theta = theta[..., :N_ROT]
    ct = jnp.cos(theta)
    st = jnp.sin(theta)
    proj = jnp.einsum('bnd,md->bnm', point_positions, rff_Omega)
    phi = jnp.concatenate([jnp.cos(proj), jnp.sin(proj)], axis=-1)
    pos_feat = jnp.einsum('bnm,pm->bnp', phi, posC_W)
    ones_bn = jnp.ones((B, N, 1), jnp.float32)
    F = jnp.concatenate(
        [ct, ct - st * st, -st, ct * st, pos_feat, ones_bn,
         jnp.zeros((B, N, NF - 81), jnp.float32)], axis=-1).astype(jnp.bfloat16)

    G = jnp.asarray(_G_STATIC).astype(jnp.bfloat16)
    gvec = jnp.concatenate(
        [jnp.zeros((N_HEADS, HEAD_DIM - POS_TAIL), jnp.float32),
         POS_SCALE * scale * pos_head_gain], axis=1).reshape(1, D)

    valid = neuron_pad_mask != 0
    spk = ((spike_mask != 0) & valid[:, None, :]).astype(jnp.float32)
    spk4 = spk.reshape(B, T, 1, N)
    validf = valid.astype(jnp.float32).reshape(B, N, 1)
    rw = rms_w.reshape(1, D)
    ones_mx = jnp.ones((N, 128), jnp.bfloat16)

    wqkv = jnp.concatenate([Wq * scale, Wk, Wv], axis=0).astype(jnp.bfloat16)
    wob = Wo.astype(jnp.bfloat16)

    out = pl.pallas_call(
        _fused_kernel,
        grid=(B, T),
        in_specs=[
            pl.BlockSpec((1, 1, N, D), lambda b, t: (b, t, 0, 0)),
            pl.BlockSpec((1, 1, 1, N), lambda b, t: (b, t, 0, 0)),
            pl.BlockSpec((1, N, 1), lambda b, t: (b, 0, 0)),
            pl.BlockSpec((1, N, NF), lambda b, t: (b, 0, 0)),
            pl.BlockSpec((NF, 4 * D), lambda b, t: (0, 0)),
            pl.BlockSpec((1, D), lambda b, t: (0, 0)),
            pl.BlockSpec((N, 128), lambda b, t: (0, 0)),
            pl.BlockSpec((1, D), lambda b, t: (0, 0)),
            pl.BlockSpec((3 * D, D), lambda b, t: (0, 0)),
            pl.BlockSpec((D, D), lambda b, t: (0, 0)),
        ],
        out_specs=pl.BlockSpec((1, 1, N, D), lambda b, t: (b, t, 0, 0)),
        out_shape=jax.ShapeDtypeStruct((B, T, N, D), jnp.float32),
        compiler_params=pltpu.CompilerParams(
            dimension_semantics=("parallel", "parallel")),
    )(x, spk4, validf, F, G, gvec, ones_mx, rw, wqkv, wob)
    return out
